# baseline (speedup 1.0000x reference)
import jax
import jax.numpy as jnp
from jax.experimental import pallas as pl
from jax.experimental.pallas import tpu as pltpu

_SLOPE = 0.01
_DN = (((0,), (0,)), ((), ()))
_ROWS = 8
_CCHUNK = 256
_SLOTS = (4, 8, 4)
_PRIME = (4, 8, 4)


def _leaky(x):
    return jnp.maximum(x, _SLOPE * x)


def _make_body(n, big_dims, tn, n_enc):
    n_tiles = [n_out // tn for (_, n_out) in big_dims]
    n_big = len(big_dims)
    small_acts = (True, True, False, True)
    big_acts = (True, True, False)

    def body(*refs):
        i = 0
        x_ref = refs[i]; i += 1
        enc = [refs[i + 3 * k:i + 3 * k + 3] for k in range(4)]
        i += 12
        small = [refs[i + 3 * k:i + 3 * k + 3] for k in range(4)]
        i += 12
        big = [refs[i + 3 * k:i + 3 * k + 3] for k in range(n_big)]
        i += 3 * n_big
        o_ref = refs[i]; i += 1
        slots = []
        for l in range(n_big):
            slots.append([refs[i + u] for u in range(_SLOTS[l])])
            i += _SLOTS[l]
        acts = [refs[i + l] for l in range(n_big - 1)]
        i += n_big - 1
        pooled_scr = refs[i]; i += 1
        sem = refs[i]

        g = pl.program_id(0)
        f32 = jnp.float32
        bf16 = jnp.bfloat16

        def dma(l, j):
            u = j % _SLOTS[l]
            return pltpu.make_async_copy(big[l][0].at[j], slots[l][u],
                                         sem.at[l, u])

        @pl.when(g == 0)
        def _prime():
            for l in range(n_big):
                for j in range(_PRIME[l]):
                    dma(l, j).start()

        @pl.when(g < n_enc)
        def _enc_step():
            (w1_ref, s1_ref, t1_ref), (w2_ref, s2_ref, t2_ref), \
                (w3_ref, s3_ref, t3_ref), (w4_ref, s4_ref, t4_ref) = enc
            w1a = jnp.concatenate([w1_ref[...] * s1_ref[...], t1_ref[...]],
                                  axis=0).astype(bf16)
            w2 = (w2_ref[...].astype(f32) * s2_ref[...]).astype(bf16)
            w3 = (w3_ref[...].astype(f32) * s3_ref[...]).astype(bf16)
            w4 = (w4_ref[...].astype(f32) * s4_ref[...]).astype(bf16)
            t2 = t2_ref[...].T
            t3 = t3_ref[...].T

            s = _ROWS * n
            x = x_ref[...].astype(bf16)
            a = jnp.concatenate([x, jnp.ones((1, s), bf16)], axis=0)
            for w, t in ((w1a, None), (w2, t2), (w3, t3)):
                a = jax.lax.dot_general(w, a, _DN, preferred_element_type=f32)
                if t is not None:
                    a = a + t
                a = _leaky(a.astype(bf16))
            chunks = []
            cout = w4.shape[1]
            for c0 in range(0, cout, _CCHUNK):
                y = jax.lax.dot_general(w4[:, c0:c0 + _CCHUNK], a, _DN,
                                        preferred_element_type=f32)
                cols = [jnp.max(y[:, r * n:(r + 1) * n], axis=1, keepdims=True)
                        for r in range(_ROWS)]
                chunks.append(jnp.concatenate(cols, axis=1))
            p = jnp.concatenate(chunks, axis=0)
            pooled_scr[pl.ds(g * _ROWS, _ROWS), :] = p.T + t4_ref[...]

        @pl.when(g == n_enc)
        def _decode():
            a = pooled_scr[...].astype(bf16)
            for k in range(4):
                wr, sr, tr = small[k]
                y = jnp.dot(a, wr[...], preferred_element_type=f32)
                y = y * sr[...] + tr[...]
                if small_acts[k]:
                    y = _leaky(y)
                a = y.astype(bf16)

            cur = a
            for l in range(n_big):
                _, s_r, t_r = big[l]
                for j in range(n_tiles[l]):
                    dma(l, j).wait()
                    y = jnp.dot(cur, slots[l][j % _SLOTS[l]][...],
                                preferred_element_type=f32)
                    if j + _PRIME[l] < n_tiles[l]:
                        dma(l, j + _PRIME[l]).start()
                    y = y * s_r[:, j * tn:(j + 1) * tn] \
                        + t_r[:, j * tn:(j + 1) * tn]
                    if big_acts[l]:
                        y = _leaky(y)
                    if l + 1 < n_big:
                        acts[l][:, j * tn:(j + 1) * tn] = y.astype(bf16)
                    else:
                        o_ref[:, j * tn:(j + 1) * tn] = y
                if l + 1 < n_big:
                    cur = acts[l][...]
    return body


def kernel(x,
           enc0_w, enc0_s, enc0_t,
           enc1_w, enc1_s, enc1_t,
           enc2_w, enc2_s, enc2_t,
           enc3_w, enc3_s, enc3_t,
           fcs0_w, fcs0_s, fcs0_t,
           fcs1_w, fcs1_s, fcs1_t,
           fcs2_w, fcs2_s, fcs2_t,
           fcs3_w, fcs3_s, fcs3_t,
           fcb0_w, fcb0_s, fcb0_t,
           fcb1_w, fcb1_s, fcb1_t,
           fcb2_w, fcb2_s, fcb2_t):
    B, C, N = x.shape
    xt2 = jnp.transpose(x, (1, 0, 2)).reshape(C, B * N)
    n_enc = B // _ROWS

    fc_big = [(fcb0_w, fcb0_s, fcb0_t), (fcb1_w, fcb1_s, fcb1_t),
              (fcb2_w, fcb2_s, fcb2_t)]
    tn = fcb0_w.shape[2]
    big_dims = [(w.shape[1], w.shape[0] * w.shape[2]) for (w, _, _) in fc_big]
    n_out = big_dims[-1][1]

    flat = [xt2]
    in_specs = [pl.BlockSpec((C, _ROWS * N),
                             lambda g: (0, jnp.minimum(g, n_enc - 1)))]
    for arr in (enc0_w, enc0_s, enc0_t, enc1_w, enc1_s, enc1_t,
                enc2_w, enc2_s, enc2_t, enc3_w, enc3_s, enc3_t,
                fcs0_w, fcs0_s, fcs0_t, fcs1_w, fcs1_s, fcs1_t,
                fcs2_w, fcs2_s, fcs2_t, fcs3_w, fcs3_s, fcs3_t):
        flat.append(arr)
        in_specs.append(pl.BlockSpec(arr.shape, lambda g: (0, 0)))
    for (w, s, t) in fc_big:
        flat += [w, s, t]
        in_specs += [pl.BlockSpec(memory_space=pl.ANY),
                     pl.BlockSpec(s.shape, lambda g: (0, 0)),
                     pl.BlockSpec(t.shape, lambda g: (0, 0))]

    scratch_shapes = []
    for l, (k_in, _) in enumerate(big_dims):
        for _u in range(_SLOTS[l]):
            scratch_shapes.append(pltpu.VMEM((k_in, tn), jnp.bfloat16))
    for (_, n_mid) in big_dims[:-1]:
        scratch_shapes.append(pltpu.VMEM((B, n_mid), jnp.bfloat16))
    scratch_shapes.append(pltpu.VMEM((B, enc3_w.shape[1]), jnp.float32))
    scratch_shapes.append(pltpu.SemaphoreType.DMA((len(big_dims),
                                                   max(_SLOTS))))

    return pl.pallas_call(
        _make_body(N, big_dims, tn, n_enc),
        out_shape=jax.ShapeDtypeStruct((B, n_out), jnp.float32),
        grid=(n_enc + 1,),
        in_specs=in_specs,
        out_specs=pl.BlockSpec((B, n_out), lambda g: (0, 0)),
        scratch_shapes=scratch_shapes,
        compiler_params=pltpu.CompilerParams(
            dimension_semantics=("arbitrary",),
            vmem_limit_bytes=64 * 1024 * 1024),
    )(*flat)


# speedup vs baseline: 1.0060x; 1.0060x over previous
import jax
import jax.numpy as jnp
from jax.experimental import pallas as pl
from jax.experimental.pallas import tpu as pltpu

_SLOPE = 0.01
_DN = (((0,), (0,)), ((), ()))
_ROWS = 8
_CCHUNK = 256
_SLOTS = (4, 8, 4)
_PRIME = (4, 8, 4)


def _leaky(x):
    return jnp.maximum(x, _SLOPE * x)


def _make_body(n, big_dims, tn, n_enc):
    n_tiles = [n_out // tn for (_, n_out) in big_dims]
    n_big = len(big_dims)
    small_acts = (True, True, False, True)
    big_acts = (True, True, False)

    def body(*refs):
        i = 0
        x_ref = refs[i]; i += 1
        enc = [refs[i + 3 * k:i + 3 * k + 3] for k in range(4)]
        i += 12
        small = [refs[i + 3 * k:i + 3 * k + 3] for k in range(4)]
        i += 12
        big = [refs[i + 3 * k:i + 3 * k + 3] for k in range(n_big)]
        i += 3 * n_big
        o_ref = refs[i]; i += 1
        slots = []
        for l in range(n_big):
            slots.append([refs[i + u] for u in range(_SLOTS[l])])
            i += _SLOTS[l]
        acts = [refs[i + l] for l in range(n_big - 1)]
        i += n_big - 1
        pooled_scr = refs[i]; i += 1
        sem = refs[i]

        g = pl.program_id(0)
        f32 = jnp.float32
        bf16 = jnp.bfloat16

        def dma(l, j):
            u = j % _SLOTS[l]
            return pltpu.make_async_copy(big[l][0].at[j], slots[l][u],
                                         sem.at[l, u])

        @pl.when(g == 0)
        def _prime():
            for l in range(n_big):
                for j in range(_PRIME[l]):
                    dma(l, j).start()

        @pl.when(g < n_enc)
        def _enc_step():
            (w1_ref, s1_ref, t1_ref), (w2_ref, s2_ref, t2_ref), \
                (w3_ref, s3_ref, t3_ref), (w4_ref, s4_ref, t4_ref) = enc
            w1a = jnp.concatenate([w1_ref[...] * s1_ref[...], t1_ref[...]],
                                  axis=0).astype(bf16)
            w2 = (w2_ref[...].astype(f32) * s2_ref[...]).astype(bf16)
            w3 = (w3_ref[...].astype(f32) * s3_ref[...]).astype(bf16)
            w4 = (w4_ref[...].astype(f32) * s4_ref[...]).astype(bf16)

            s = _ROWS * n
            x = x_ref[...].astype(bf16)
            xa = jnp.concatenate([x, jnp.ones((1, s), bf16)], axis=0)
            a = jax.lax.dot_general(xa, w1a, _DN,
                                    preferred_element_type=f32)
            a = _leaky(a.astype(bf16))
            a = jnp.dot(a, w2, preferred_element_type=f32) + t2_ref[...]
            a = _leaky(a.astype(bf16))
            a = jnp.dot(a, w3, preferred_element_type=f32) + t3_ref[...]
            a = _leaky(a.astype(bf16))
            chunks = []
            cout = w4.shape[1]
            for c0 in range(0, cout, _CCHUNK):
                y = jnp.dot(a, w4[:, c0:c0 + _CCHUNK],
                            preferred_element_type=f32)
                rows = [jnp.max(y[r * n:(r + 1) * n, :], axis=0, keepdims=True)
                        for r in range(_ROWS)]
                chunks.append(jnp.concatenate(rows, axis=0))
            p = jnp.concatenate(chunks, axis=1)
            pooled_scr[pl.ds(g * _ROWS, _ROWS), :] = p + t4_ref[...]

        @pl.when(g == n_enc)
        def _decode():
            a = pooled_scr[...].astype(bf16)
            for k in range(4):
                wr, sr, tr = small[k]
                y = jnp.dot(a, wr[...], preferred_element_type=f32)
                y = y * sr[...] + tr[...]
                if small_acts[k]:
                    y = _leaky(y)
                a = y.astype(bf16)

            cur = a
            for l in range(n_big):
                _, s_r, t_r = big[l]
                for j in range(n_tiles[l]):
                    dma(l, j).wait()
                    y = jnp.dot(cur, slots[l][j % _SLOTS[l]][...],
                                preferred_element_type=f32)
                    if j + _PRIME[l] < n_tiles[l]:
                        dma(l, j + _PRIME[l]).start()
                    y = y * s_r[:, j * tn:(j + 1) * tn] \
                        + t_r[:, j * tn:(j + 1) * tn]
                    if big_acts[l]:
                        y = _leaky(y)
                    if l + 1 < n_big:
                        acts[l][:, j * tn:(j + 1) * tn] = y.astype(bf16)
                    else:
                        o_ref[:, j * tn:(j + 1) * tn] = y
                if l + 1 < n_big:
                    cur = acts[l][...]
    return body


def kernel(x,
           enc0_w, enc0_s, enc0_t,
           enc1_w, enc1_s, enc1_t,
           enc2_w, enc2_s, enc2_t,
           enc3_w, enc3_s, enc3_t,
           fcs0_w, fcs0_s, fcs0_t,
           fcs1_w, fcs1_s, fcs1_t,
           fcs2_w, fcs2_s, fcs2_t,
           fcs3_w, fcs3_s, fcs3_t,
           fcb0_w, fcb0_s, fcb0_t,
           fcb1_w, fcb1_s, fcb1_t,
           fcb2_w, fcb2_s, fcb2_t):
    B, C, N = x.shape
    xt2 = jnp.transpose(x, (1, 0, 2)).reshape(C, B * N)
    n_enc = B // _ROWS

    fc_big = [(fcb0_w, fcb0_s, fcb0_t), (fcb1_w, fcb1_s, fcb1_t),
              (fcb2_w, fcb2_s, fcb2_t)]
    tn = fcb0_w.shape[2]
    big_dims = [(w.shape[1], w.shape[0] * w.shape[2]) for (w, _, _) in fc_big]
    n_out = big_dims[-1][1]

    flat = [xt2]
    in_specs = [pl.BlockSpec((C, _ROWS * N),
                             lambda g: (0, jnp.minimum(g, n_enc - 1)))]
    for arr in (enc0_w, enc0_s, enc0_t, enc1_w, enc1_s, enc1_t,
                enc2_w, enc2_s, enc2_t, enc3_w, enc3_s, enc3_t,
                fcs0_w, fcs0_s, fcs0_t, fcs1_w, fcs1_s, fcs1_t,
                fcs2_w, fcs2_s, fcs2_t, fcs3_w, fcs3_s, fcs3_t):
        flat.append(arr)
        in_specs.append(pl.BlockSpec(arr.shape, lambda g: (0, 0)))
    for (w, s, t) in fc_big:
        flat += [w, s, t]
        in_specs += [pl.BlockSpec(memory_space=pl.ANY),
                     pl.BlockSpec(s.shape, lambda g: (0, 0)),
                     pl.BlockSpec(t.shape, lambda g: (0, 0))]

    scratch_shapes = []
    for l, (k_in, _) in enumerate(big_dims):
        for _u in range(_SLOTS[l]):
            scratch_shapes.append(pltpu.VMEM((k_in, tn), jnp.bfloat16))
    for (_, n_mid) in big_dims[:-1]:
        scratch_shapes.append(pltpu.VMEM((B, n_mid), jnp.bfloat16))
    scratch_shapes.append(pltpu.VMEM((B, enc3_w.shape[1]), jnp.float32))
    scratch_shapes.append(pltpu.SemaphoreType.DMA((len(big_dims),
                                                   max(_SLOTS))))

    return pl.pallas_call(
        _make_body(N, big_dims, tn, n_enc),
        out_shape=jax.ShapeDtypeStruct((B, n_out), jnp.float32),
        grid=(n_enc + 1,),
        in_specs=in_specs,
        out_specs=pl.BlockSpec((B, n_out), lambda g: (0, 0)),
        scratch_shapes=scratch_shapes,
        compiler_params=pltpu.CompilerParams(
            dimension_semantics=("arbitrary",),
            vmem_limit_bytes=64 * 1024 * 1024),
    )(*flat)
